# Initial kernel scaffold; baseline (speedup 1.0000x reference)
import jax
import jax.numpy as jnp
from jax.experimental import pallas as pl


def kernel(cls_outputs, box_outputs, anchor_boxes, indices, img_scale, img_size):
    raise NotImplementedError("write your pallas kernel here")


# speedup vs baseline: 1.6425x; 1.6425x over previous
import jax
import jax.numpy as jnp
from jax.experimental import pallas as pl
from jax.experimental.pallas import tpu as pltpu

N_BOXES = 20000
NUM_CLASSES = 90
MIN_SCORE = 0.05
IOU_THRESHOLD = 0.5
MAX_DET = 200
TOP_K = 2000
NMS_ITERS = 5

_ROW_CHUNK = 400


def _score_body(cls_ref, conf_ref, class_ref):
    s = jax.nn.sigmoid(cls_ref[...])
    conf = jnp.max(s, axis=1, keepdims=True)
    lane = jax.lax.broadcasted_iota(jnp.int32, s.shape, 1)
    cls_idx = jnp.min(jnp.where(s == conf, lane, NUM_CLASSES), axis=1,
                      keepdims=True)
    conf_ref[...] = jnp.where(conf >= MIN_SCORE, conf, -1.0)
    class_ref[...] = cls_idx


def _scores(cls_outputs):
    return pl.pallas_call(
        _score_body,
        out_shape=[
            jax.ShapeDtypeStruct((N_BOXES, 1), jnp.float32),
            jax.ShapeDtypeStruct((N_BOXES, 1), jnp.int32),
        ],
    )(cls_outputs)


def _nms_body(codes_ref, anch_ref, codes_t_ref, anch_t_ref, conf_ref,
              sz_ref, sz_t_ref, b_ref, keep_ref, m_ref, mt_ref):
    anch = anch_ref[...]
    codes = codes_ref[...]
    ycenter_a = (anch[:, 0:1] + anch[:, 2:3]) / 2.0
    xcenter_a = (anch[:, 1:2] + anch[:, 3:4]) / 2.0
    ha = anch[:, 2:3] - anch[:, 0:1]
    wa = anch[:, 3:4] - anch[:, 1:2]
    ty = codes[:, 0:1]
    tx = codes[:, 1:2]
    th = codes[:, 2:3]
    tw = codes[:, 3:4]
    w = jnp.exp(tw) * wa
    h = jnp.exp(th) * ha
    yc = ty * ha + ycenter_a
    xc = tx * wa + xcenter_a
    b = jnp.concatenate(
        [xc - w / 2.0, yc - h / 2.0, xc + w / 2.0, yc + h / 2.0], axis=1)
    b = jnp.minimum(jnp.maximum(b, 0.0), sz_ref[...])
    b_ref[...] = b
    x0 = b[:, 0:1]
    y0 = b[:, 1:2]
    x1 = b[:, 2:3]
    y1 = b[:, 3:4]
    area = (x1 - x0) * (y1 - y0)
    cx = (x0 + x1) / 2.0
    cy = (y0 + y1) / 2.0

    anch_t = anch_t_ref[...]
    codes_t = codes_t_ref[...]
    ycenter_at = (anch_t[0:1, :] + anch_t[2:3, :]) / 2.0
    xcenter_at = (anch_t[1:2, :] + anch_t[3:4, :]) / 2.0
    hat = anch_t[2:3, :] - anch_t[0:1, :]
    wat = anch_t[3:4, :] - anch_t[1:2, :]
    tyt = codes_t[0:1, :]
    txt = codes_t[1:2, :]
    tht = codes_t[2:3, :]
    twt = codes_t[3:4, :]
    wt = jnp.exp(twt) * wat
    ht = jnp.exp(tht) * hat
    yct = tyt * hat + ycenter_at
    xct = txt * wat + xcenter_at
    bt = jnp.concatenate(
        [xct - wt / 2.0, yct - ht / 2.0, xct + wt / 2.0, yct + ht / 2.0],
        axis=0)
    bt = jnp.minimum(jnp.maximum(bt, 0.0), sz_t_ref[...])
    x0t = bt[0:1, :]
    y0t = bt[1:2, :]
    x1t = bt[2:3, :]
    y1t = bt[3:4, :]
    area_t = (x1t - x0t) * (y1t - y0t)
    cxt = (x0t + x1t) / 2.0
    cyt = (y0t + y1t) / 2.0

    R = _ROW_CHUNK
    cols = jax.lax.broadcasted_iota(jnp.int32, (R, TOP_K), 1)
    rows_base = jax.lax.broadcasted_iota(jnp.int32, (R, TOP_K), 0)

    for i in range(TOP_K // R):
        r0 = i * R
        x0i = x0[r0:r0 + R, :]
        y0i = y0[r0:r0 + R, :]
        x1i = x1[r0:r0 + R, :]
        y1i = y1[r0:r0 + R, :]
        ai = area[r0:r0 + R, :]
        cxi = cx[r0:r0 + R, :]
        cyi = cy[r0:r0 + R, :]
        wx = jnp.maximum(jnp.minimum(x1i, x1t) - jnp.maximum(x0i, x0t), 0.0)
        wy = jnp.maximum(jnp.minimum(y1i, y1t) - jnp.maximum(y0i, y0t), 0.0)
        inter = wx * wy
        union = ai + area_t - inter
        iou = inter / jnp.maximum(union, 1e-8)
        d2 = (cxi - cxt) ** 2 + (cyi - cyt) ** 2
        ex = jnp.maximum(x1i, x1t) - jnp.minimum(x0i, x0t)
        ey = jnp.maximum(y1i, y1t) - jnp.minimum(y0i, y0t)
        c2 = ex ** 2 + ey ** 2
        d = iou - d2 / jnp.maximum(c2, 1e-8)
        rows = rows_base + r0
        m_ref[r0:r0 + R, :] = jnp.where(rows < cols, d, 0.0)
        mt_ref[r0:r0 + R, :] = jnp.where(cols < rows, d, 0.0)

    m = m_ref[...]
    mt = mt_ref[...]
    e_col = jnp.ones((TOP_K, 1), jnp.float32)
    e_row = jnp.ones((1, TOP_K), jnp.float32)
    for _ in range(NMS_ITERS):
        max_col = jnp.max(mt * e_row, axis=1, keepdims=True)
        max_row = jnp.max(m * e_col, axis=0, keepdims=True)
        e_col = (max_col <= IOU_THRESHOLD).astype(jnp.float32)
        e_row = (max_row <= IOU_THRESHOLD).astype(jnp.float32)
    max_col = jnp.max(mt * e_row, axis=1, keepdims=True)
    keep = jnp.logical_and(max_col <= IOU_THRESHOLD,
                           conf_ref[...] >= MIN_SCORE)
    keep_ref[...] = keep.astype(jnp.float32)


def _nms(codes, anch, conf, sz):
    return pl.pallas_call(
        _nms_body,
        out_shape=[
            jax.ShapeDtypeStruct((TOP_K, 4), jnp.float32),
            jax.ShapeDtypeStruct((TOP_K, 1), jnp.float32),
        ],
        scratch_shapes=[
            pltpu.VMEM((TOP_K, TOP_K), jnp.float32),
            pltpu.VMEM((TOP_K, TOP_K), jnp.float32),
        ],
        compiler_params=pltpu.CompilerParams(
            vmem_limit_bytes=100 * 1024 * 1024),
    )(codes, anch, codes.T, anch.T, conf, sz, sz.T)


def kernel(cls_outputs, box_outputs, anchor_boxes, indices, img_scale, img_size):
    conf_m, classes = _scores(cls_outputs.astype(jnp.float32))
    conf_m = conf_m.reshape(N_BOXES)
    classes = classes.reshape(N_BOXES)

    c, order = jax.lax.top_k(conf_m, TOP_K)

    codes = box_outputs.astype(jnp.float32)[order]
    anch = anchor_boxes[indices[order]]
    cls_sel = classes[order]

    size = img_size / img_scale
    sz = jnp.concatenate([size, size], axis=0).reshape(1, 4)

    b, keep_f = _nms(codes, anch, c.reshape(TOP_K, 1), sz)
    keep = keep_f.reshape(TOP_K) > 0.5

    rank_key = jnp.where(keep, c + 1.0, 0.0)
    _, rank = jax.lax.top_k(rank_key, MAX_DET)
    kb = b[rank]
    ks = c[rank]
    kc = cls_sel[rank]
    kk = keep[rank]
    bw = kb[:, 2] - kb[:, 0]
    bh = kb[:, 3] - kb[:, 1]
    out_boxes = jnp.stack([kb[:, 0], kb[:, 1], bw, bh], axis=1) * img_scale
    det = jnp.concatenate(
        [out_boxes, ks[:, None], (kc[:, None] + 1).astype(jnp.float32)],
        axis=1)
    return jnp.where(kk[:, None], det, 0.0)


# speedup vs baseline: 1.6669x; 1.0148x over previous
import jax
import jax.numpy as jnp
from jax.experimental import pallas as pl
from jax.experimental.pallas import tpu as pltpu

N_BOXES = 20000
NUM_CLASSES = 90
MIN_SCORE = 0.05
IOU_THRESHOLD = 0.5
MAX_DET = 200
TOP_K = 2000
NMS_ITERS = 5

_ROW_CHUNK = 400


def _score_body(cls_ref, conf_ref, class_ref):
    s = jax.nn.sigmoid(cls_ref[...])
    conf = jnp.max(s, axis=1, keepdims=True)
    lane = jax.lax.broadcasted_iota(jnp.int32, s.shape, 1)
    cls_idx = jnp.min(jnp.where(s == conf, lane, NUM_CLASSES), axis=1,
                      keepdims=True)
    conf_ref[...] = jnp.where(conf >= MIN_SCORE, conf, -1.0)
    class_ref[...] = cls_idx


_SCORE_BLOCK = 2000


def _scores(cls_outputs):
    return pl.pallas_call(
        _score_body,
        grid=(N_BOXES // _SCORE_BLOCK,),
        in_specs=[pl.BlockSpec((_SCORE_BLOCK, NUM_CLASSES), lambda i: (i, 0))],
        out_specs=[
            pl.BlockSpec((_SCORE_BLOCK, 1), lambda i: (i, 0)),
            pl.BlockSpec((_SCORE_BLOCK, 1), lambda i: (i, 0)),
        ],
        out_shape=[
            jax.ShapeDtypeStruct((N_BOXES, 1), jnp.float32),
            jax.ShapeDtypeStruct((N_BOXES, 1), jnp.int32),
        ],
    )(cls_outputs)


def _nms_body(codes_ref, anch_ref, codes_t_ref, anch_t_ref, conf_ref,
              sz_ref, sz_t_ref, b_ref, keep_ref, m_ref):
    anch = anch_ref[...]
    codes = codes_ref[...]
    ycenter_a = (anch[:, 0:1] + anch[:, 2:3]) / 2.0
    xcenter_a = (anch[:, 1:2] + anch[:, 3:4]) / 2.0
    ha = anch[:, 2:3] - anch[:, 0:1]
    wa = anch[:, 3:4] - anch[:, 1:2]
    ty = codes[:, 0:1]
    tx = codes[:, 1:2]
    th = codes[:, 2:3]
    tw = codes[:, 3:4]
    w = jnp.exp(tw) * wa
    h = jnp.exp(th) * ha
    yc = ty * ha + ycenter_a
    xc = tx * wa + xcenter_a
    b = jnp.concatenate(
        [xc - w / 2.0, yc - h / 2.0, xc + w / 2.0, yc + h / 2.0], axis=1)
    b = jnp.minimum(jnp.maximum(b, 0.0), sz_ref[...])
    b_ref[...] = b
    x0 = b[:, 0:1]
    y0 = b[:, 1:2]
    x1 = b[:, 2:3]
    y1 = b[:, 3:4]
    area = (x1 - x0) * (y1 - y0)
    cx = (x0 + x1) / 2.0
    cy = (y0 + y1) / 2.0

    anch_t = anch_t_ref[...]
    codes_t = codes_t_ref[...]
    ycenter_at = (anch_t[0:1, :] + anch_t[2:3, :]) / 2.0
    xcenter_at = (anch_t[1:2, :] + anch_t[3:4, :]) / 2.0
    hat = anch_t[2:3, :] - anch_t[0:1, :]
    wat = anch_t[3:4, :] - anch_t[1:2, :]
    tyt = codes_t[0:1, :]
    txt = codes_t[1:2, :]
    tht = codes_t[2:3, :]
    twt = codes_t[3:4, :]
    wt = jnp.exp(twt) * wat
    ht = jnp.exp(tht) * hat
    yct = tyt * hat + ycenter_at
    xct = txt * wat + xcenter_at
    bt = jnp.concatenate(
        [xct - wt / 2.0, yct - ht / 2.0, xct + wt / 2.0, yct + ht / 2.0],
        axis=0)
    bt = jnp.minimum(jnp.maximum(bt, 0.0), sz_t_ref[...])
    x0t = bt[0:1, :]
    y0t = bt[1:2, :]
    x1t = bt[2:3, :]
    y1t = bt[3:4, :]
    area_t = (x1t - x0t) * (y1t - y0t)
    cxt = (x0t + x1t) / 2.0
    cyt = (y0t + y1t) / 2.0

    R = _ROW_CHUNK
    cols = jax.lax.broadcasted_iota(jnp.int32, (R, TOP_K), 1)
    rows_base = jax.lax.broadcasted_iota(jnp.int32, (R, TOP_K), 0)

    for i in range(TOP_K // R):
        r0 = i * R
        x0i = x0[r0:r0 + R, :]
        y0i = y0[r0:r0 + R, :]
        x1i = x1[r0:r0 + R, :]
        y1i = y1[r0:r0 + R, :]
        ai = area[r0:r0 + R, :]
        cxi = cx[r0:r0 + R, :]
        cyi = cy[r0:r0 + R, :]
        wx = jnp.maximum(jnp.minimum(x1i, x1t) - jnp.maximum(x0i, x0t), 0.0)
        wy = jnp.maximum(jnp.minimum(y1i, y1t) - jnp.maximum(y0i, y0t), 0.0)
        inter = wx * wy
        union = ai + area_t - inter
        iou = inter / jnp.maximum(union, 1e-8)
        d2 = (cxi - cxt) ** 2 + (cyi - cyt) ** 2
        ex = jnp.maximum(x1i, x1t) - jnp.minimum(x0i, x0t)
        ey = jnp.maximum(y1i, y1t) - jnp.minimum(y0i, y0t)
        c2 = ex ** 2 + ey ** 2
        d = iou - d2 / jnp.maximum(c2, 1e-8)
        rows = rows_base + r0
        m_ref[r0:r0 + R, :] = jnp.where(rows < cols, d, 0.0)

    m = m_ref[...]
    e_col = jnp.ones((TOP_K, 1), jnp.float32)
    for _ in range(NMS_ITERS):
        max_row = jnp.max(m * e_col, axis=0, keepdims=True)
        e_col = jnp.transpose(
            (max_row <= IOU_THRESHOLD).astype(jnp.float32))
    max_row = jnp.max(m * e_col, axis=0, keepdims=True)
    keep = jnp.logical_and(jnp.transpose(max_row) <= IOU_THRESHOLD,
                           conf_ref[...] >= MIN_SCORE)
    keep_ref[...] = keep.astype(jnp.float32)


def _nms(codes, anch, conf, sz):
    return pl.pallas_call(
        _nms_body,
        out_shape=[
            jax.ShapeDtypeStruct((TOP_K, 4), jnp.float32),
            jax.ShapeDtypeStruct((TOP_K, 1), jnp.float32),
        ],
        scratch_shapes=[
            pltpu.VMEM((TOP_K, TOP_K), jnp.float32),
        ],
        compiler_params=pltpu.CompilerParams(
            vmem_limit_bytes=100 * 1024 * 1024),
    )(codes, anch, codes.T, anch.T, conf, sz, sz.T)


def kernel(cls_outputs, box_outputs, anchor_boxes, indices, img_scale, img_size):
    conf_m, classes = _scores(cls_outputs.astype(jnp.float32))
    conf_m = conf_m.reshape(N_BOXES)
    classes = classes.reshape(N_BOXES)

    c, order = jax.lax.top_k(conf_m, TOP_K)

    codes = box_outputs.astype(jnp.float32)[order]
    anch = anchor_boxes[indices[order]]
    cls_sel = classes[order]

    size = img_size / img_scale
    sz = jnp.concatenate([size, size], axis=0).reshape(1, 4)

    b, keep_f = _nms(codes, anch, c.reshape(TOP_K, 1), sz)
    keep = keep_f.reshape(TOP_K) > 0.5

    rank_key = jnp.where(keep, c + 1.0, 0.0)
    _, rank = jax.lax.top_k(rank_key, MAX_DET)
    kb = b[rank]
    ks = c[rank]
    kc = cls_sel[rank]
    kk = keep[rank]
    bw = kb[:, 2] - kb[:, 0]
    bh = kb[:, 3] - kb[:, 1]
    out_boxes = jnp.stack([kb[:, 0], kb[:, 1], bw, bh], axis=1) * img_scale
    det = jnp.concatenate(
        [out_boxes, ks[:, None], (kc[:, None] + 1).astype(jnp.float32)],
        axis=1)
    return jnp.where(kk[:, None], det, 0.0)


# speedup vs baseline: 1.7491x; 1.0493x over previous
import jax
import jax.numpy as jnp
from jax.experimental import pallas as pl
from jax.experimental.pallas import tpu as pltpu

N_BOXES = 20000
NUM_CLASSES = 90
MIN_SCORE = 0.05
IOU_THRESHOLD = 0.5
MAX_DET = 200
TOP_K = 2000
NMS_ITERS = 5

_ROW_CHUNK = 400


def _score_body(cls_ref, conf_ref, class_ref):
    s = jax.nn.sigmoid(cls_ref[...])
    conf = jnp.max(s, axis=1, keepdims=True)
    lane = jax.lax.broadcasted_iota(jnp.int32, s.shape, 1)
    cls_idx = jnp.min(jnp.where(s == conf, lane, NUM_CLASSES), axis=1,
                      keepdims=True)
    conf_ref[...] = jnp.where(conf >= MIN_SCORE, conf, -1.0)
    class_ref[...] = cls_idx


_SCORE_BLOCK = 2000


def _scores(cls_outputs):
    return pl.pallas_call(
        _score_body,
        grid=(N_BOXES // _SCORE_BLOCK,),
        in_specs=[pl.BlockSpec((_SCORE_BLOCK, NUM_CLASSES), lambda i: (i, 0))],
        out_specs=[
            pl.BlockSpec((_SCORE_BLOCK, 1), lambda i: (i, 0)),
            pl.BlockSpec((_SCORE_BLOCK, 1), lambda i: (i, 0)),
        ],
        out_shape=[
            jax.ShapeDtypeStruct((N_BOXES, 1), jnp.float32),
            jax.ShapeDtypeStruct((N_BOXES, 1), jnp.int32),
        ],
    )(cls_outputs)


def _nms_body(codes_ref, anch_ref, codes_t_ref, anch_t_ref, conf_ref,
              conf_t_ref, clsp1_ref, sz_ref, sz_t_ref, scale_ref,
              det_ref, m_ref):
    anch = anch_ref[...]
    codes = codes_ref[...]
    ycenter_a = (anch[:, 0:1] + anch[:, 2:3]) / 2.0
    xcenter_a = (anch[:, 1:2] + anch[:, 3:4]) / 2.0
    ha = anch[:, 2:3] - anch[:, 0:1]
    wa = anch[:, 3:4] - anch[:, 1:2]
    ty = codes[:, 0:1]
    tx = codes[:, 1:2]
    th = codes[:, 2:3]
    tw = codes[:, 3:4]
    w = jnp.exp(tw) * wa
    h = jnp.exp(th) * ha
    yc = ty * ha + ycenter_a
    xc = tx * wa + xcenter_a
    b = jnp.concatenate(
        [xc - w / 2.0, yc - h / 2.0, xc + w / 2.0, yc + h / 2.0], axis=1)
    b = jnp.minimum(jnp.maximum(b, 0.0), sz_ref[...])
    x0 = b[:, 0:1]
    y0 = b[:, 1:2]
    x1 = b[:, 2:3]
    y1 = b[:, 3:4]
    area = (x1 - x0) * (y1 - y0)
    cx = (x0 + x1) / 2.0
    cy = (y0 + y1) / 2.0

    anch_t = anch_t_ref[...]
    codes_t = codes_t_ref[...]
    ycenter_at = (anch_t[0:1, :] + anch_t[2:3, :]) / 2.0
    xcenter_at = (anch_t[1:2, :] + anch_t[3:4, :]) / 2.0
    hat = anch_t[2:3, :] - anch_t[0:1, :]
    wat = anch_t[3:4, :] - anch_t[1:2, :]
    tyt = codes_t[0:1, :]
    txt = codes_t[1:2, :]
    tht = codes_t[2:3, :]
    twt = codes_t[3:4, :]
    wt = jnp.exp(twt) * wat
    ht = jnp.exp(tht) * hat
    yct = tyt * hat + ycenter_at
    xct = txt * wat + xcenter_at
    bt = jnp.concatenate(
        [xct - wt / 2.0, yct - ht / 2.0, xct + wt / 2.0, yct + ht / 2.0],
        axis=0)
    bt = jnp.minimum(jnp.maximum(bt, 0.0), sz_t_ref[...])
    x0t = bt[0:1, :]
    y0t = bt[1:2, :]
    x1t = bt[2:3, :]
    y1t = bt[3:4, :]
    area_t = (x1t - x0t) * (y1t - y0t)
    cxt = (x0t + x1t) / 2.0
    cyt = (y0t + y1t) / 2.0

    R = _ROW_CHUNK
    cols = jax.lax.broadcasted_iota(jnp.int32, (R, TOP_K), 1)
    rows_base = jax.lax.broadcasted_iota(jnp.int32, (R, TOP_K), 0)

    for i in range(TOP_K // R):
        r0 = i * R
        x0i = x0[r0:r0 + R, :]
        y0i = y0[r0:r0 + R, :]
        x1i = x1[r0:r0 + R, :]
        y1i = y1[r0:r0 + R, :]
        ai = area[r0:r0 + R, :]
        cxi = cx[r0:r0 + R, :]
        cyi = cy[r0:r0 + R, :]
        wx = jnp.maximum(jnp.minimum(x1i, x1t) - jnp.maximum(x0i, x0t), 0.0)
        wy = jnp.maximum(jnp.minimum(y1i, y1t) - jnp.maximum(y0i, y0t), 0.0)
        inter = wx * wy
        union = ai + area_t - inter
        iou = inter / jnp.maximum(union, 1e-8)
        d2 = (cxi - cxt) ** 2 + (cyi - cyt) ** 2
        ex = jnp.maximum(x1i, x1t) - jnp.minimum(x0i, x0t)
        ey = jnp.maximum(y1i, y1t) - jnp.minimum(y0i, y0t)
        c2 = ex ** 2 + ey ** 2
        d = iou - d2 / jnp.maximum(c2, 1e-8)
        rows = rows_base + r0
        m_ref[r0:r0 + R, :] = jnp.where(rows < cols, d, 0.0)

    m = m_ref[...]
    e_col = jnp.ones((TOP_K, 1), jnp.float32)
    for _ in range(NMS_ITERS):
        max_row = jnp.max(m * e_col, axis=0, keepdims=True)
        e_col = jnp.transpose(
            (max_row <= IOU_THRESHOLD).astype(jnp.float32))
    max_row = jnp.max(m * e_col, axis=0, keepdims=True)
    keep_row = jnp.logical_and(max_row <= IOU_THRESHOLD,
                               conf_t_ref[...] >= MIN_SCORE)

    kf = keep_row.astype(jnp.float32)
    tri = jnp.where(
        jax.lax.broadcasted_iota(jnp.int32, (TOP_K, TOP_K), 0)
        <= jax.lax.broadcasted_iota(jnp.int32, (TOP_K, TOP_K), 1),
        1.0, 0.0)
    pos = jax.lax.dot_general(
        kf, tri, (((1,), (0,)), ((), ())),
        preferred_element_type=jnp.float32) - 1.0
    pos_i = pos.astype(jnp.int32)
    slot = jax.lax.broadcasted_iota(jnp.int32, (MAX_DET, TOP_K), 0)
    onehot = jnp.where(jnp.logical_and(slot == pos_i, keep_row), 1.0, 0.0)
    s = scale_ref[...]
    det_rows = jnp.concatenate(
        [x0 * s, y0 * s, (x1 - x0) * s, (y1 - y0) * s,
         conf_ref[...], clsp1_ref[...]], axis=1)
    det_ref[...] = jax.lax.dot_general(
        onehot, det_rows, (((1,), (0,)), ((), ())),
        precision=jax.lax.Precision.HIGHEST,
        preferred_element_type=jnp.float32)


def _nms(codes, anch, conf, clsp1, sz, scale):
    return pl.pallas_call(
        _nms_body,
        out_shape=jax.ShapeDtypeStruct((MAX_DET, 6), jnp.float32),
        scratch_shapes=[
            pltpu.VMEM((TOP_K, TOP_K), jnp.float32),
        ],
        compiler_params=pltpu.CompilerParams(
            vmem_limit_bytes=100 * 1024 * 1024),
    )(codes, anch, codes.T, anch.T, conf, conf.T, clsp1, sz, sz.T, scale)


def kernel(cls_outputs, box_outputs, anchor_boxes, indices, img_scale, img_size):
    conf_m, classes = _scores(cls_outputs.astype(jnp.float32))
    conf_m = conf_m.reshape(N_BOXES)

    c, order = jax.lax.top_k(conf_m, TOP_K)

    codes = box_outputs.astype(jnp.float32)[order]
    anch = anchor_boxes[indices[order]]
    clsp1 = (classes.reshape(N_BOXES)[order] + 1).astype(jnp.float32)

    size = img_size / img_scale
    sz = jnp.concatenate([size, size], axis=0).reshape(1, 4)

    return _nms(codes, anch, c.reshape(TOP_K, 1), clsp1.reshape(TOP_K, 1),
                sz, img_scale.reshape(1, 1))
